# baseline (speedup 1.0000x reference)
import functools

import jax
import jax.numpy as jnp
from jax import lax
from jax.experimental import pallas as pl
from jax.experimental.pallas import tpu as pltpu
from jax.experimental.pallas import tpu_sc as plsc

B, L, D = 16, 2048, 128
NC, NS = 2, 16
NW = NC * NS
CW = L // NW
NCHUNK = B
LANES = 16
NBUF = 8

_mesh = plsc.VectorSubcoreMesh(core_axis_name="c", subcore_axis_name="s")


@functools.partial(
    pl.kernel,
    out_type=jax.ShapeDtypeStruct((B, L, D), jnp.float32),
    mesh=_mesh,
    scratch_types=[
        pltpu.VMEM((NCHUNK, CW), jnp.int32),
        pltpu.VMEM((CW, D), jnp.float32),
        pltpu.VMEM((NBUF, CW, D), jnp.float32),
        pltpu.SemaphoreType.DMA((NBUF,)),
        pltpu.SemaphoreType.DMA((NBUF,)),
    ],
)
def _tok_pos(x_hbm, txt_hbm, pos_hbm, out_hbm, idx_v, pos_v, rows_v, gsem, osem):
    wid = lax.axis_index("s") * NC + lax.axis_index("c")
    col = wid * CW
    pltpu.sync_copy(x_hbm.at[wid], idx_v)
    pltpu.sync_copy(pos_hbm.at[pl.ds(col, CW)], pos_v)

    def gather(j, bb):
        return pltpu.async_copy(
            txt_hbm.at[idx_v.at[j]], rows_v.at[bb], gsem.at[bb])

    def writeback(j, bb):
        return pltpu.async_copy(
            rows_v.at[bb], out_hbm.at[j, pl.ds(col, CW)], osem.at[bb])

    def add_pos(bb):
        rv = rows_v.at[bb]

        def row_body(r, carry):
            for t in range(D // LANES):
                sl = pl.ds(t * LANES, LANES)
                rv[r, sl] = rv[r, sl] + pos_v[r, sl]
            return carry

        lax.fori_loop(0, CW, row_body, 0)

    for b in range(NBUF):
        gather(b, b)

    def loop_body(j, carry):
        bb = lax.rem(j, NBUF)
        jn = j + NBUF - 1

        @pl.when(jnp.logical_and(j >= 1, jn < NCHUNK))
        def _issue_ahead():
            bp = lax.rem(jn, NBUF)
            pltpu.make_async_copy(
                rows_v.at[bp], out_hbm.at[0, pl.ds(col, CW)], osem.at[bp]).wait()
            gather(jn, bp)

        pltpu.make_async_copy(
            txt_hbm.at[idx_v.at[0]], rows_v.at[bb], gsem.at[bb]).wait()
        add_pos(bb)
        writeback(j, bb)
        return carry

    lax.fori_loop(0, NCHUNK, loop_body, 0)

    for b in range(NBUF):
        pltpu.make_async_copy(
            rows_v.at[b], out_hbm.at[0, pl.ds(col, CW)], osem.at[b]).wait()


def kernel(x, txt_emb, pos_emb):
    xr = x.reshape(B, NW, CW).swapaxes(0, 1).astype(jnp.int32)
    return _tok_pos(xr, txt_emb, pos_emb)


# speedup vs baseline: 1.8428x; 1.8428x over previous
import functools

import jax
import jax.numpy as jnp
from jax import lax
from jax.experimental import pallas as pl
from jax.experimental.pallas import tpu as pltpu
from jax.experimental.pallas import tpu_sc as plsc

B, L, D = 16, 2048, 128
NC, NS = 2, 16
NW = NC * NS
CW = L // NW
NCHUNK = B
LANES = 16
NBUF = 8

_mesh = plsc.VectorSubcoreMesh(core_axis_name="c", subcore_axis_name="s")


@functools.partial(
    pl.kernel,
    out_type=jax.ShapeDtypeStruct((B, L, D), jnp.float32),
    mesh=_mesh,
    scratch_types=[
        pltpu.VMEM((NCHUNK, CW), jnp.int32),
        pltpu.VMEM((CW, D), jnp.float32),
        pltpu.VMEM((NBUF, CW, D), jnp.float32),
        pltpu.SemaphoreType.DMA((NBUF,)),
        pltpu.SemaphoreType.DMA((NBUF,)),
    ],
)
def _tok_pos(x_hbm, txt_hbm, pos_hbm, out_hbm, idx_v, pos_v, rows_v, gsem, osem):
    wid = lax.axis_index("s") * NC + lax.axis_index("c")
    col = wid * CW
    pltpu.sync_copy(x_hbm.at[wid], idx_v)
    pltpu.sync_copy(pos_hbm.at[pl.ds(col, CW)], pos_v)

    def gather(j):
        bb = j % NBUF
        return pltpu.async_copy(
            txt_hbm.at[idx_v.at[j]], rows_v.at[bb], gsem.at[bb])

    def writeback(j):
        bb = j % NBUF
        return pltpu.async_copy(
            rows_v.at[bb], out_hbm.at[j, pl.ds(col, CW)], osem.at[bb])

    def add_pos(bb):
        rv = rows_v.at[bb]

        def row_body(r, carry):
            for t in range(D // LANES):
                sl = pl.ds(t * LANES, LANES)
                rv[r, sl] = rv[r, sl] + pos_v[r, sl]
            return carry

        lax.fori_loop(0, CW, row_body, 0)

    gat, out = {}, {}
    for j in range(NBUF):
        gat[j] = gather(j)
    for j in range(NCHUNK):
        if j >= 1 and (j - 1) + NBUF < NCHUNK:
            out[j - 1].wait()
            gat[j - 1 + NBUF] = gather(j - 1 + NBUF)
        gat[j].wait()
        add_pos(j % NBUF)
        out[j] = writeback(j)
    for j in range(NCHUNK - NBUF, NCHUNK):
        out[j].wait()


def kernel(x, txt_emb, pos_emb):
    xr = x.reshape(B, NW, CW).swapaxes(0, 1).astype(jnp.int32)
    return _tok_pos(xr, txt_emb, pos_emb)
